# Initial kernel scaffold; baseline (speedup 1.0000x reference)
import jax
import jax.numpy as jnp
from jax.experimental import pallas as pl


def kernel(user_id, item_id, user_review, user_review_scores, item_reviews, item_review_scores, user_conv_w, user_conv_b, item_conv_w, item_conv_b, user_prop_pref_tab, item_prop_pref_tab, user_bias_tab, item_bias_tab, mu_bias, user_emb_tab, item_emb_tab):
    raise NotImplementedError("write your pallas kernel here")


# speedup vs baseline: 1.3082x; 1.3082x over previous
import jax
import jax.numpy as jnp
from jax.experimental import pallas as pl
from jax.experimental.pallas import tpu as pltpu

B = 256
C = 40
P = 768
H = 15
F = 600
FPAD = 640
CB = 8
KSTEPS = C // CB


def _toeplitz(conv_w):
    shifted = [jnp.pad(conv_w, ((0, 0), (0, 0), (h, (P - conv_w.shape[2]) - h)))
               for h in range(H)]
    wb = jnp.stack(shifted, axis=1)
    wb = wb.reshape(F, C * P)
    wb = jnp.concatenate([wb, jnp.zeros((FPAD - F, C * P), conv_w.dtype)], axis=0)
    return wb.T


def _conv_body(rev_ref, sc_ref, pref_ref, wt_ref, b_ref, out_ref, acc_ref):
    k = pl.program_id(1)
    rev = rev_ref[0]
    sc = sc_ref[0]
    pref = pref_ref[0]
    w = jnp.sum(sc * pref[:, None, :], axis=2) * (1.0 / 6.0)
    re = (rev * w[:, :, None]).reshape(B, CB * P)
    part = jax.lax.dot_general(re, wt_ref[0], (((1,), (0,)), ((), ())),
                               preferred_element_type=jnp.float32)

    @pl.when(k == 0)
    def _():
        acc_ref[...] = part

    @pl.when(k > 0)
    def _():
        acc_ref[...] += part

    @pl.when(k == pl.num_programs(1) - 1)
    def _():
        out_ref[0] = jnp.maximum(acc_ref[...] + b_ref[0], 0.0)


def _combine_body(f_ref, ub_ref, ib_ref, mu_ref, o_ref):
    s = jnp.sum(f_ref[0] * f_ref[1], axis=1, keepdims=True)
    o_ref[...] = s + ub_ref[...] + ib_ref[...] + mu_ref[0, 0]


def kernel(user_id, item_id, user_review, user_review_scores, item_reviews,
           item_review_scores, user_conv_w, user_conv_b, item_conv_w,
           item_conv_b, user_prop_pref_tab, item_prop_pref_tab, user_bias_tab,
           item_bias_tab, mu_bias, user_emb_tab, item_emb_tab):
    pref = jnp.stack([user_prop_pref_tab[user_id], item_prop_pref_tab[item_id]])
    reviews = jnp.stack([user_review, item_reviews])
    scores = jnp.stack([user_review_scores.transpose(0, 2, 1),
                        item_review_scores.transpose(0, 2, 1)])
    wts = jnp.stack([_toeplitz(user_conv_w), _toeplitz(item_conv_w)])
    bias = jnp.stack([
        jnp.pad(jnp.repeat(user_conv_b, H), (0, FPAD - F))[None, :],
        jnp.pad(jnp.repeat(item_conv_b, H), (0, FPAD - F))[None, :],
    ])

    feats = pl.pallas_call(
        _conv_body,
        grid=(2, KSTEPS),
        in_specs=[
            pl.BlockSpec((1, B, CB, P), lambda t, k: (t, 0, k, 0)),
            pl.BlockSpec((1, B, CB, 6), lambda t, k: (t, 0, k, 0)),
            pl.BlockSpec((1, B, 6), lambda t, k: (t, 0, 0)),
            pl.BlockSpec((1, CB * P, FPAD), lambda t, k: (t, k, 0)),
            pl.BlockSpec((1, 1, FPAD), lambda t, k: (t, 0, 0)),
        ],
        out_specs=pl.BlockSpec((1, B, FPAD), lambda t, k: (t, 0, 0)),
        out_shape=jax.ShapeDtypeStruct((2, B, FPAD), jnp.float32),
        scratch_shapes=[pltpu.VMEM((B, FPAD), jnp.float32)],
    )(reviews, scores, pref, wts, bias)

    out = pl.pallas_call(
        _combine_body,
        in_specs=[
            pl.BlockSpec((2, B, FPAD), lambda: (0, 0, 0)),
            pl.BlockSpec((B, 1), lambda: (0, 0)),
            pl.BlockSpec((B, 1), lambda: (0, 0)),
            pl.BlockSpec((1, 1), lambda: (0, 0)),
        ],
        out_specs=pl.BlockSpec((B, 1), lambda: (0, 0)),
        out_shape=jax.ShapeDtypeStruct((B, 1), jnp.float32),
    )(feats, user_bias_tab[user_id], item_bias_tab[item_id],
      mu_bias.reshape(1, 1))
    return out[:, 0]


# speedup vs baseline: 2.7367x; 2.0919x over previous
import jax
import jax.numpy as jnp
from jax.experimental import pallas as pl
from jax.experimental.pallas import tpu as pltpu

B = 256
BB = 128
C = 40
P = 768
H = 15
F = 600
CB = 8
KSTEPS = C // CB
BSTEPS = B // BB


def _toeplitz_t(conv_w):
    kw = conv_w.shape[2]
    cw = conv_w.astype(jnp.bfloat16).transpose(1, 2, 0)
    shifted = [jnp.pad(cw, ((0, 0), (h, (P - kw) - h), (0, 0))) for h in range(H)]
    wt = jnp.stack(shifted, axis=3)
    return wt.reshape(C * P, F)


def _body(ru_ref, ri_ref, wtu_ref, wti_ref, wu_ref, wi_ref, bu_ref, bi_ref,
          ub_ref, ib_ref, mu_ref, o_ref, au_ref, ai_ref):
    k = pl.program_id(0)
    b = pl.program_id(1)
    rows = pl.ds(b * BB, BB)

    def branch(rev_ref, wt_ref, w_ref, acc_ref):
        w = w_ref[...].T
        re = (rev_ref[...] * w[:, :, None]).astype(jnp.bfloat16).reshape(BB, CB * P)
        part = jax.lax.dot_general(re, wt_ref[...], (((1,), (0,)), ((), ())),
                                   preferred_element_type=jnp.float32)

        @pl.when(k == 0)
        def _():
            acc_ref[rows, :] = part

        @pl.when(k > 0)
        def _():
            acc_ref[rows, :] += part

    branch(ru_ref, wtu_ref, wu_ref, au_ref)
    branch(ri_ref, wti_ref, wi_ref, ai_ref)

    @pl.when(k == pl.num_programs(0) - 1)
    def _():
        fu = jnp.maximum(au_ref[rows, :] + bu_ref[...], 0.0)
        fi = jnp.maximum(ai_ref[rows, :] + bi_ref[...], 0.0)
        s = jnp.sum(fu * fi, axis=1, keepdims=True)
        o_ref[rows, :] = s + ub_ref[rows, :] + ib_ref[rows, :] + mu_ref[0, 0]


def kernel(user_id, item_id, user_review, user_review_scores, item_reviews,
           item_review_scores, user_conv_w, user_conv_b, item_conv_w,
           item_conv_b, user_prop_pref_tab, item_prop_pref_tab, user_bias_tab,
           item_bias_tab, mu_bias, user_emb_tab, item_emb_tab):
    wu = jnp.einsum('imj,im->ji', user_review_scores,
                    user_prop_pref_tab[user_id]) * (1.0 / 6.0)
    wi = jnp.einsum('imj,im->ji', item_review_scores,
                    item_prop_pref_tab[item_id]) * (1.0 / 6.0)
    out = pl.pallas_call(
        _body,
        grid=(KSTEPS, BSTEPS),
        in_specs=[
            pl.BlockSpec((BB, CB, P), lambda k, b: (b, k, 0)),
            pl.BlockSpec((BB, CB, P), lambda k, b: (b, k, 0)),
            pl.BlockSpec((CB * P, F), lambda k, b: (k, 0)),
            pl.BlockSpec((CB * P, F), lambda k, b: (k, 0)),
            pl.BlockSpec((CB, BB), lambda k, b: (k, b)),
            pl.BlockSpec((CB, BB), lambda k, b: (k, b)),
            pl.BlockSpec((1, F), lambda k, b: (0, 0)),
            pl.BlockSpec((1, F), lambda k, b: (0, 0)),
            pl.BlockSpec((B, 1), lambda k, b: (0, 0)),
            pl.BlockSpec((B, 1), lambda k, b: (0, 0)),
            pl.BlockSpec((1, 1), lambda k, b: (0, 0)),
        ],
        out_specs=pl.BlockSpec((B, 1), lambda k, b: (0, 0)),
        out_shape=jax.ShapeDtypeStruct((B, 1), jnp.float32),
        scratch_shapes=[pltpu.VMEM((B, F), jnp.float32),
                        pltpu.VMEM((B, F), jnp.float32)],
    )(user_review, item_reviews,
      _toeplitz_t(user_conv_w), _toeplitz_t(item_conv_w),
      wu, wi,
      jnp.repeat(user_conv_b, H)[None, :], jnp.repeat(item_conv_b, H)[None, :],
      user_bias_tab[user_id], item_bias_tab[item_id], mu_bias.reshape(1, 1))
    return out[:, 0]
